# initial kernel scaffold (unmeasured)
import jax
import jax.numpy as jnp
from jax import lax
from jax.experimental import pallas as pl
from jax.experimental.pallas import tpu as pltpu

N_DEV = 32
E_LOCAL = 4
N_TOK = 2048
D = 512
H = 1024
ROWS = N_TOK // N_DEV


def kernel(x, router_W, route_idx, expert_W, shared_W):
    k = lax.axis_index("i")

    scores = jnp.dot(x, router_W, preferred_element_type=jnp.float32)
    probs = jax.nn.softmax(scores, axis=-1)
    p = jnp.take_along_axis(probs, route_idx, axis=1)[:, 0]
    e = route_idx[:, 0]
    local_j = e - E_LOCAL * k
    oh = (local_j[:, None] == jnp.arange(E_LOCAL)[None, :])
    gate = oh.astype(jnp.float32) * p[:, None]

    xm = (gate[:, :, None] * x[:, None, :]).reshape(N_TOK, E_LOCAL * D)
    w_cat = expert_W.reshape(E_LOCAL * D, H)

    shared_own = jnp.dot(
        lax.dynamic_slice(x, (ROWS * k, 0), (ROWS, D)),
        shared_W,
        preferred_element_type=jnp.float32,
    )

    def body(xm_ref, w_ref, sh_ref, out_ref,
             partial_ref, acc_ref, recv_ref, send_sems, recv_sems):
        my = lax.axis_index("i")
        left = (my - 1 + N_DEV) % N_DEV
        right = (my + 1) % N_DEV

        barrier_sem = pltpu.get_barrier_semaphore()
        for nbr in (left, right):
            pl.semaphore_signal(
                barrier_sem, inc=1,
                device_id=(nbr,), device_id_type=pl.DeviceIdType.MESH,
            )
        pl.semaphore_wait(barrier_sem, 2)

        partial_ref[...] = jnp.dot(
            xm_ref[...], w_ref[...], preferred_element_type=jnp.float32
        )

        c0 = (my - 1 + N_DEV) % N_DEV
        acc_ref[0, :, :] = partial_ref[pl.ds(c0 * ROWS, ROWS), :]
        for s in range(N_DEV - 1):
            slot = s % 2
            rdma = pltpu.make_async_remote_copy(
                src_ref=acc_ref.at[slot],
                dst_ref=recv_ref.at[s],
                send_sem=send_sems.at[s],
                recv_sem=recv_sems.at[s],
                device_id=(right,),
                device_id_type=pl.DeviceIdType.MESH,
            )
            rdma.start()
            rdma.wait()
            if s < N_DEV - 2:
                c = (my - s - 2 + N_DEV) % N_DEV
                acc_ref[1 - slot, :, :] = (
                    recv_ref[s] + partial_ref[pl.ds(c * ROWS, ROWS), :]
                )
            else:
                out_ref[...] = (
                    recv_ref[s]
                    + partial_ref[pl.ds(my * ROWS, ROWS), :]
                    + sh_ref[...]
                )

    return pl.pallas_call(
        body,
        out_shape=jax.ShapeDtypeStruct((ROWS, H), jnp.float32),
        in_specs=[
            pl.BlockSpec(memory_space=pltpu.VMEM),
            pl.BlockSpec(memory_space=pltpu.VMEM),
            pl.BlockSpec(memory_space=pltpu.VMEM),
        ],
        out_specs=pl.BlockSpec(memory_space=pltpu.VMEM),
        scratch_shapes=[
            pltpu.VMEM((N_TOK, H), jnp.float32),
            pltpu.VMEM((2, ROWS, H), jnp.float32),
            pltpu.VMEM((N_DEV - 1, ROWS, H), jnp.float32),
            pltpu.SemaphoreType.DMA((N_DEV - 1,)),
            pltpu.SemaphoreType.DMA((N_DEV - 1,)),
        ],
        compiler_params=pltpu.CompilerParams(collective_id=0),
    )(xm, w_cat, shared_own)


# baseline (device time: 189071 ns/iter reference)
import jax
import jax.numpy as jnp
from jax import lax
from jax.experimental import pallas as pl
from jax.experimental.pallas import tpu as pltpu

N_DEV = 32
E_LOCAL = 4
N_TOK = 2048
D = 512
H = 1024
ROWS = N_TOK // N_DEV


def kernel(x, router_W, route_idx, expert_W, shared_W):
    k = lax.axis_index("i")

    scores = jnp.dot(x, router_W, preferred_element_type=jnp.float32)
    probs = jax.nn.softmax(scores, axis=-1)
    p = jnp.take_along_axis(probs, route_idx, axis=1)[:, 0]
    e = route_idx[:, 0]
    local_j = e - E_LOCAL * k
    oh = (local_j[:, None] == jnp.arange(E_LOCAL)[None, :])
    gate = oh.astype(jnp.float32) * p[:, None]

    shared_own = jnp.dot(
        lax.dynamic_slice(x, (ROWS * k, 0), (ROWS, D)),
        shared_W,
        preferred_element_type=jnp.float32,
    )

    def body(x_ref, gate_ref, w_ref, sh_ref, out_ref,
             acc_ref, recv_ref, send_sems, recv_sems):
        my = lax.axis_index("i")
        left = (my - 1 + N_DEV) % N_DEV
        right = (my + 1) % N_DEV

        barrier_sem = pltpu.get_barrier_semaphore()
        for nbr in (left, right):
            pl.semaphore_signal(
                barrier_sem, inc=1,
                device_id=(nbr,), device_id_type=pl.DeviceIdType.MESH,
            )
        pl.semaphore_wait(barrier_sem, 2)

        def chunk_partial(c):
            xrow = x_ref[pl.ds(c * ROWS, ROWS), :]
            g = gate_ref[pl.ds(c * ROWS, ROWS), :]
            acc = jnp.zeros((ROWS, H), jnp.float32)
            for j in range(E_LOCAL):
                acc = acc + jnp.dot(
                    xrow * g[:, j:j + 1], w_ref[j],
                    preferred_element_type=jnp.float32,
                )
            return acc

        c0 = (my - 1 + N_DEV) % N_DEV
        acc_ref[0, :, :] = chunk_partial(c0)
        for s in range(N_DEV - 1):
            slot = s % 2
            rdma = pltpu.make_async_remote_copy(
                src_ref=acc_ref.at[slot],
                dst_ref=recv_ref.at[s],
                send_sem=send_sems.at[s],
                recv_sem=recv_sems.at[s],
                device_id=(right,),
                device_id_type=pl.DeviceIdType.MESH,
            )
            rdma.start()
            rdma.wait()
            if s < N_DEV - 2:
                c = (my - s - 2 + N_DEV) % N_DEV
                acc_ref[1 - slot, :, :] = recv_ref[s] + chunk_partial(c)
            else:
                out_ref[...] = recv_ref[s] + chunk_partial(my) + sh_ref[...]

    return pl.pallas_call(
        body,
        out_shape=jax.ShapeDtypeStruct((ROWS, H), jnp.float32),
        in_specs=[
            pl.BlockSpec(memory_space=pltpu.VMEM),
            pl.BlockSpec(memory_space=pltpu.VMEM),
            pl.BlockSpec(memory_space=pltpu.VMEM),
            pl.BlockSpec(memory_space=pltpu.VMEM),
        ],
        out_specs=pl.BlockSpec(memory_space=pltpu.VMEM),
        scratch_shapes=[
            pltpu.VMEM((2, ROWS, H), jnp.float32),
            pltpu.VMEM((N_DEV - 1, ROWS, H), jnp.float32),
            pltpu.SemaphoreType.DMA((N_DEV - 1,)),
            pltpu.SemaphoreType.DMA((N_DEV - 1,)),
        ],
        compiler_params=pltpu.CompilerParams(collective_id=0),
    )(x, gate, expert_W, shared_own)


# device time: 160429 ns/iter; 1.1785x vs baseline; 1.1785x over previous
import jax
import jax.numpy as jnp
from jax import lax
from jax.experimental import pallas as pl
from jax.experimental.pallas import tpu as pltpu

N_DEV = 32
E_LOCAL = 4
N_TOK = 2048
D = 512
H = 1024
ROWS = N_TOK // N_DEV


def kernel(x, router_W, route_idx, expert_W, shared_W):
    k = lax.axis_index("i")

    scores = jnp.dot(x, router_W, preferred_element_type=jnp.float32)
    probs = jax.nn.softmax(scores, axis=-1)
    oh128 = (jnp.arange(probs.shape[1])[None, :] == route_idx)
    p = jnp.sum(probs * oh128.astype(jnp.float32), axis=1)
    e = route_idx[:, 0]
    local_j = e - E_LOCAL * k
    oh = (local_j[:, None] == jnp.arange(E_LOCAL)[None, :])
    gate = oh.astype(jnp.float32) * p[:, None]

    shared_own = jnp.dot(
        lax.dynamic_slice(x, (ROWS * k, 0), (ROWS, D)),
        shared_W,
        preferred_element_type=jnp.float32,
    )

    def body(x_ref, gate_ref, w_ref, sh_ref, out_ref,
             acc_ref, recv_ref, send_sems, recv_sems):
        my = lax.axis_index("i")
        left = (my - 1 + N_DEV) % N_DEV
        right = (my + 1) % N_DEV

        barrier_sem = pltpu.get_barrier_semaphore()
        for nbr in (left, right):
            pl.semaphore_signal(
                barrier_sem, inc=1,
                device_id=(nbr,), device_id_type=pl.DeviceIdType.MESH,
            )
        pl.semaphore_wait(barrier_sem, 2)

        def chunk_partial(c):
            xrow = x_ref[pl.ds(c * ROWS, ROWS), :]
            g = gate_ref[pl.ds(c * ROWS, ROWS), :]
            acc = jnp.zeros((ROWS, H), jnp.float32)
            for j in range(E_LOCAL):
                acc = acc + jnp.dot(
                    xrow * g[:, j:j + 1], w_ref[j],
                    preferred_element_type=jnp.float32,
                )
            return acc

        c0 = (my - 1 + N_DEV) % N_DEV
        acc_ref[0, :, :] = chunk_partial(c0)
        for s in range(N_DEV - 1):
            slot = s % 2
            rdma = pltpu.make_async_remote_copy(
                src_ref=acc_ref.at[slot],
                dst_ref=recv_ref.at[s],
                send_sem=send_sems.at[s],
                recv_sem=recv_sems.at[s],
                device_id=(right,),
                device_id_type=pl.DeviceIdType.MESH,
            )
            rdma.start()
            if s < N_DEV - 2:
                c = (my - s - 2 + N_DEV) % N_DEV
                part = chunk_partial(c)
                rdma.wait()
                acc_ref[1 - slot, :, :] = recv_ref[s] + part
            else:
                part = chunk_partial(my) + sh_ref[...]
                rdma.wait()
                out_ref[...] = recv_ref[s] + part

    return pl.pallas_call(
        body,
        out_shape=jax.ShapeDtypeStruct((ROWS, H), jnp.float32),
        in_specs=[
            pl.BlockSpec(memory_space=pltpu.VMEM),
            pl.BlockSpec(memory_space=pltpu.VMEM),
            pl.BlockSpec(memory_space=pltpu.VMEM),
            pl.BlockSpec(memory_space=pltpu.VMEM),
        ],
        out_specs=pl.BlockSpec(memory_space=pltpu.VMEM),
        scratch_shapes=[
            pltpu.VMEM((2, ROWS, H), jnp.float32),
            pltpu.VMEM((N_DEV - 1, ROWS, H), jnp.float32),
            pltpu.SemaphoreType.DMA((N_DEV - 1,)),
            pltpu.SemaphoreType.DMA((N_DEV - 1,)),
        ],
        compiler_params=pltpu.CompilerParams(collective_id=0),
    )(x, gate, expert_W, shared_own)
